# Initial kernel scaffold; baseline (speedup 1.0000x reference)
import jax
import jax.numpy as jnp
from jax.experimental import pallas as pl


def kernel(x, table):
    raise NotImplementedError("write your pallas kernel here")


# speedup vs baseline: 1.6711x; 1.6711x over previous
import jax
import jax.numpy as jnp
from jax.experimental import pallas as pl

SEQ_BLK = 1024


def _add_kernel(x_ref, t_ref, o_ref):
    o_ref[...] = x_ref[...] + t_ref[...]


def kernel(x, table):
    B, S, D = x.shape
    grid = (S // SEQ_BLK, B)
    return pl.pallas_call(
        _add_kernel,
        grid=grid,
        in_specs=[
            pl.BlockSpec((1, SEQ_BLK, D), lambda i, j: (j, i, 0)),
            pl.BlockSpec((SEQ_BLK, D), lambda i, j: (i, 0)),
        ],
        out_specs=pl.BlockSpec((1, SEQ_BLK, D), lambda i, j: (j, i, 0)),
        out_shape=jax.ShapeDtypeStruct(x.shape, x.dtype),
    )(x, table)
